# baseline (speedup 1.0000x reference)
import functools

import jax
import jax.numpy as jnp
from jax import lax
from jax.experimental import pallas as pl
from jax.experimental.pallas import tpu as pltpu
from jax.experimental.pallas import tpu_sc as plsc

N_NODES = 10000
K_NEI = 16
D_FEAT = 128
L_LAYERS = 4

_NC = 2
_NS = 16
_NW = _NC * _NS

_CH_SIZES = (800, 2400, 2400, 2400, 2000)
_CH_STARTS = (0, 800, 3200, 5600, 8000)
_CHUNK = 40
_GROUP = 5


@functools.lru_cache(maxsize=None)
def _make_sc_gather(nc):
    mesh = plsc.VectorSubcoreMesh(core_axis_name="c", subcore_axis_name="s")
    per_w = nc * K_NEI // _NW
    n_chunks = per_w // _CHUNK
    n_groups = n_chunks // _GROUP
    tot_chunks = nc * K_NEI // _CHUNK
    nslot = min(4, n_groups)

    @functools.partial(
        pl.kernel,
        mesh=mesh,
        out_type=jax.ShapeDtypeStruct((tot_chunks, _CHUNK, D_FEAT), jnp.float32),
        scratch_types=[
            pltpu.VMEM((n_chunks, _CHUNK), jnp.int32),
            pltpu.VMEM((nslot, _GROUP, _CHUNK, D_FEAT), jnp.float32),
            [pltpu.SemaphoreType.DMA] * nslot,
        ],
    )
    def gather_kernel(table_hbm, idx_hbm, out_hbm, idx_v, rows_v, sems):
        wid = lax.axis_index("s") * _NC + lax.axis_index("c")
        cbase = wid * n_chunks
        pltpu.sync_copy(idx_hbm.at[wid], idx_v)

        def fire(g):
            s = g % nslot
            return [
                pltpu.async_copy(
                    table_hbm.at[idx_v.at[g * _GROUP + j]],
                    rows_v.at[s].at[j], sems[s])
                for j in range(_GROUP)
            ]

        handles = {g: fire(g) for g in range(nslot)}
        for g in range(n_groups):
            for h in handles.pop(g):
                h.wait()
            pltpu.sync_copy(rows_v.at[g % nslot],
                            out_hbm.at[pl.ds(cbase + g * _GROUP, _GROUP)])
            if g + nslot < n_groups:
                handles[g + nslot] = fire(g + nslot)

    return gather_kernel


def _make_lstm_body(bn, relu):
    def body(m_ref, h_ref, wcat_ref, b_ref, ws_ref, wn_ref, bo_ref, out_ref):
        hqb = jnp.zeros((bn, D_FEAT), jnp.bfloat16)
        cp = jnp.zeros((bn, D_FEAT), jnp.float32)
        for t in range(K_NEI):
            x_cat = jnp.concatenate(
                [m_ref[t].astype(jnp.bfloat16), hqb], axis=1)
            g = jnp.dot(x_cat, wcat_ref[...],
                        preferred_element_type=jnp.float32) + b_ref[...]
            ti1 = jnp.tanh(g[:, :D_FEAT]) + 1.0
            tf1 = jnp.tanh(g[:, D_FEAT:2 * D_FEAT]) + 1.0
            gg = jnp.tanh(g[:, 2 * D_FEAT:3 * D_FEAT])
            to1 = jnp.tanh(g[:, 3 * D_FEAT:]) + 1.0
            if t == 0:
                cp = 0.5 * (gg * ti1)
            else:
                cp = 0.5 * (cp * tf1 + gg * ti1)
            hq = jnp.tanh(cp) * to1
            hqb = hq.astype(jnp.bfloat16)

        out = (
            jnp.dot(h_ref[...].astype(jnp.bfloat16), ws_ref[...],
                    preferred_element_type=jnp.float32)
            + jnp.dot(hqb, wn_ref[...], preferred_element_type=jnp.float32)
            + bo_ref[...]
        )
        if relu:
            out = jnp.maximum(out, 0.0)
        out_ref[...] = out

    return body


@functools.lru_cache(maxsize=None)
def _make_tc_layer(nc, start, relu, bn=400):
    grid = (nc // bn,)
    full = lambda j: (0, 0)
    coff = start // bn
    return pl.pallas_call(
        _make_lstm_body(bn, relu),
        grid=grid,
        in_specs=[
            pl.BlockSpec((K_NEI, bn, D_FEAT), lambda j: (0, j, 0)),
            pl.BlockSpec((bn, D_FEAT), lambda j: (coff + j, 0)),
            pl.BlockSpec((2 * D_FEAT, 4 * D_FEAT), full),
            pl.BlockSpec((1, 4 * D_FEAT), full),
            pl.BlockSpec((D_FEAT, D_FEAT), full),
            pl.BlockSpec((D_FEAT, D_FEAT), full),
            pl.BlockSpec((1, D_FEAT), full),
        ],
        out_specs=pl.BlockSpec((bn, D_FEAT), lambda j: (j, 0)),
        out_shape=jax.ShapeDtypeStruct((nc, D_FEAT), jnp.float32),
        compiler_params=pltpu.CompilerParams(
            dimension_semantics=("parallel",)),
    )


def kernel(x, edge_index, W_ih, W_hh, b_ih, b_hh, W_self, b_self, W_neigh, b_neigh):
    src = edge_index[0]
    src_r = src.reshape(N_NODES, K_NEI)
    idxs = [
        jnp.transpose(src_r[s:s + nc]).reshape(_NW, nc * K_NEI // _NW // _CHUNK,
                                               _CHUNK)
        for nc, s in zip(_CH_SIZES, _CH_STARTS)
    ]

    cs = jnp.concatenate([
        jnp.full((D_FEAT,), 0.5, jnp.float32),
        jnp.full((D_FEAT,), 0.5, jnp.float32),
        jnp.ones((D_FEAT,), jnp.float32),
        jnp.full((D_FEAT,), 0.5, jnp.float32),
    ])
    wihT = jnp.transpose(W_ih, (0, 2, 1)) * cs
    whhT = jnp.transpose(W_hh, (0, 2, 1)) * (0.5 * cs)
    wcat = jnp.concatenate([wihT, whhT], axis=1).astype(jnp.bfloat16)
    b2 = ((b_ih + b_hh) * cs).reshape(L_LAYERS, 1, 4 * D_FEAT)
    wsT = jnp.transpose(W_self, (0, 2, 1)).astype(jnp.bfloat16)
    wnT = (0.5 * jnp.transpose(W_neigh, (0, 2, 1))).astype(jnp.bfloat16)
    bo2 = (b_self + b_neigh).reshape(L_LAYERS, 1, D_FEAT)

    h = x
    for l in range(L_LAYERS):
        relu = l < L_LAYERS - 1
        ms = [_make_sc_gather(nc)(h, idxs[c])
              for c, nc in enumerate(_CH_SIZES)]
        outs = []
        for c, (nc, s) in enumerate(zip(_CH_SIZES, _CH_STARTS)):
            m_knd = ms[c].reshape(K_NEI, nc, D_FEAT)
            outs.append(_make_tc_layer(nc, s, relu)(
                m_knd, h, wcat[l], b2[l], wsT[l], wnT[l], bo2[l]))
        h = jnp.concatenate(outs, axis=0)
    return h


# speedup vs baseline: 1.0141x; 1.0141x over previous
import functools

import jax
import jax.numpy as jnp
from jax import lax
from jax.experimental import pallas as pl
from jax.experimental.pallas import tpu as pltpu
from jax.experimental.pallas import tpu_sc as plsc

N_NODES = 10000
K_NEI = 16
D_FEAT = 128
L_LAYERS = 4

_NC = 2
_NS = 16
_NW = _NC * _NS

_CH_SIZES = (2000, 2000, 2000, 2000, 2000)
_CH_STARTS = (0, 2000, 4000, 6000, 8000)
_CHUNK = 40
_GROUP = 5


@functools.lru_cache(maxsize=None)
def _make_sc_gather(nc):
    mesh = plsc.VectorSubcoreMesh(core_axis_name="c", subcore_axis_name="s")
    per_w = nc * K_NEI // _NW
    n_chunks = per_w // _CHUNK
    n_groups = n_chunks // _GROUP
    tot_chunks = nc * K_NEI // _CHUNK
    nslot = min(4, n_groups)

    @functools.partial(
        pl.kernel,
        mesh=mesh,
        out_type=jax.ShapeDtypeStruct((tot_chunks, _CHUNK, D_FEAT), jnp.float32),
        scratch_types=[
            pltpu.VMEM((n_chunks, _CHUNK), jnp.int32),
            pltpu.VMEM((nslot, _GROUP, _CHUNK, D_FEAT), jnp.float32),
            [pltpu.SemaphoreType.DMA] * nslot,
        ],
    )
    def gather_kernel(table_hbm, idx_hbm, out_hbm, idx_v, rows_v, sems):
        wid = lax.axis_index("s") * _NC + lax.axis_index("c")
        cbase = wid * n_chunks
        pltpu.sync_copy(idx_hbm.at[wid], idx_v)

        def fire(g):
            s = g % nslot
            return [
                pltpu.async_copy(
                    table_hbm.at[idx_v.at[g * _GROUP + j]],
                    rows_v.at[s].at[j], sems[s])
                for j in range(_GROUP)
            ]

        handles = {g: fire(g) for g in range(nslot)}
        for g in range(n_groups):
            for h in handles.pop(g):
                h.wait()
            pltpu.sync_copy(rows_v.at[g % nslot],
                            out_hbm.at[pl.ds(cbase + g * _GROUP, _GROUP)])
            if g + nslot < n_groups:
                handles[g + nslot] = fire(g + nslot)

    return gather_kernel


def _make_lstm_body(bn, relu):
    def body(m_ref, h_ref, wcat_ref, b_ref, ws_ref, wn_ref, bo_ref, out_ref):
        hqb = jnp.zeros((bn, D_FEAT), jnp.bfloat16)
        cp = jnp.zeros((bn, D_FEAT), jnp.float32)
        for t in range(K_NEI):
            x_cat = jnp.concatenate(
                [m_ref[t].astype(jnp.bfloat16), hqb], axis=1)
            g = jnp.dot(x_cat, wcat_ref[...],
                        preferred_element_type=jnp.float32) + b_ref[...]
            ti1 = jnp.tanh(g[:, :D_FEAT]) + 1.0
            tf1 = jnp.tanh(g[:, D_FEAT:2 * D_FEAT]) + 1.0
            gg = jnp.tanh(g[:, 2 * D_FEAT:3 * D_FEAT])
            to1 = jnp.tanh(g[:, 3 * D_FEAT:]) + 1.0
            if t == 0:
                cp = 0.5 * (gg * ti1)
            else:
                cp = 0.5 * (cp * tf1 + gg * ti1)
            hq = jnp.tanh(cp) * to1
            hqb = hq.astype(jnp.bfloat16)

        out = (
            jnp.dot(h_ref[...].astype(jnp.bfloat16), ws_ref[...],
                    preferred_element_type=jnp.float32)
            + jnp.dot(hqb, wn_ref[...], preferred_element_type=jnp.float32)
            + bo_ref[...]
        )
        if relu:
            out = jnp.maximum(out, 0.0)
        out_ref[...] = out

    return body


@functools.lru_cache(maxsize=None)
def _make_tc_layer(nc, start, relu, bn=400):
    grid = (nc // bn,)
    full = lambda j: (0, 0)
    coff = start // bn
    return pl.pallas_call(
        _make_lstm_body(bn, relu),
        grid=grid,
        in_specs=[
            pl.BlockSpec((K_NEI, bn, D_FEAT), lambda j: (0, j, 0)),
            pl.BlockSpec((bn, D_FEAT), lambda j: (coff + j, 0)),
            pl.BlockSpec((2 * D_FEAT, 4 * D_FEAT), full),
            pl.BlockSpec((1, 4 * D_FEAT), full),
            pl.BlockSpec((D_FEAT, D_FEAT), full),
            pl.BlockSpec((D_FEAT, D_FEAT), full),
            pl.BlockSpec((1, D_FEAT), full),
        ],
        out_specs=pl.BlockSpec((bn, D_FEAT), lambda j: (j, 0)),
        out_shape=jax.ShapeDtypeStruct((nc, D_FEAT), jnp.float32),
        compiler_params=pltpu.CompilerParams(
            dimension_semantics=("parallel",)),
    )


def kernel(x, edge_index, W_ih, W_hh, b_ih, b_hh, W_self, b_self, W_neigh, b_neigh):
    src = edge_index[0]
    src_r = src.reshape(N_NODES, K_NEI)
    idxs = [
        jnp.transpose(src_r[s:s + nc]).reshape(_NW, nc * K_NEI // _NW // _CHUNK,
                                               _CHUNK)
        for nc, s in zip(_CH_SIZES, _CH_STARTS)
    ]

    cs = jnp.concatenate([
        jnp.full((D_FEAT,), 0.5, jnp.float32),
        jnp.full((D_FEAT,), 0.5, jnp.float32),
        jnp.ones((D_FEAT,), jnp.float32),
        jnp.full((D_FEAT,), 0.5, jnp.float32),
    ])
    wihT = jnp.transpose(W_ih, (0, 2, 1)) * cs
    whhT = jnp.transpose(W_hh, (0, 2, 1)) * (0.5 * cs)
    wcat = jnp.concatenate([wihT, whhT], axis=1).astype(jnp.bfloat16)
    b2 = ((b_ih + b_hh) * cs).reshape(L_LAYERS, 1, 4 * D_FEAT)
    wsT = jnp.transpose(W_self, (0, 2, 1)).astype(jnp.bfloat16)
    wnT = (0.5 * jnp.transpose(W_neigh, (0, 2, 1))).astype(jnp.bfloat16)
    bo2 = (b_self + b_neigh).reshape(L_LAYERS, 1, D_FEAT)

    h = x
    for l in range(L_LAYERS):
        relu = l < L_LAYERS - 1
        ms = [_make_sc_gather(nc)(h, idxs[c])
              for c, nc in enumerate(_CH_SIZES)]
        outs = []
        for c, (nc, s) in enumerate(zip(_CH_SIZES, _CH_STARTS)):
            m_knd = ms[c].reshape(K_NEI, nc, D_FEAT)
            outs.append(_make_tc_layer(nc, s, relu)(
                m_knd, h, wcat[l], b2[l], wsT[l], wnT[l], bo2[l]))
        h = jnp.concatenate(outs, axis=0)
    return h
